# baseline (speedup 1.0000x reference)
import functools

import jax
import jax.numpy as jnp
from jax.experimental import pallas as pl

N_NODES = 10000
N_EDGES = 320000
DELTA = 4.0
ROW_BLK = 400


def _dense_body(amax_ref, amin_ref, s1_ref, s2_ref, cnt_ref, w_ref, b_ref,
                g_ref, be_ref, out_ref):
    amax = amax_ref[...]
    amin = amin_ref[...]
    s1 = s1_ref[...]
    s2 = s2_ref[...]
    cnt = cnt_ref[...]
    mean = s1 / cnt
    var = jnp.maximum(s2 / cnt - mean * mean, 0.0)
    std = jnp.sqrt(var + 1e-5)
    aggs = jnp.concatenate([amax, amin, std, var], axis=1)
    logd = jnp.log(cnt + 1.0)
    t1 = logd / DELTA
    t2 = DELTA / logd
    scaled = jnp.concatenate([aggs, aggs * t1[:, :1], aggs * t2[:, :1]], axis=1)
    h = jax.lax.dot_general(scaled, w_ref[...], (((1,), (0,)), ((), ())),
                            preferred_element_type=jnp.float32)
    h = h + b_ref[...]
    mu = jnp.mean(h, axis=-1, keepdims=True)
    v = jnp.var(h, axis=-1, keepdims=True)
    h = (h - mu) / jnp.sqrt(v + 1e-5) * g_ref[...] + be_ref[...]
    out_ref[...] = jnp.where(h > 0, h, jnp.exp(h) - 1.0)


@functools.partial(jax.jit, static_argnames=())
def _dense_layer(amax, amin, s1, s2, cnt2d, W, b, g, be):
    grid = (N_NODES // ROW_BLK,)
    node_spec = pl.BlockSpec((ROW_BLK, 128), lambda i: (i, 0))
    return pl.pallas_call(
        _dense_body,
        grid=grid,
        in_specs=[
            node_spec, node_spec, node_spec, node_spec, node_spec,
            pl.BlockSpec((W.shape[0], 128), lambda i: (0, 0)),
            pl.BlockSpec((1, 128), lambda i: (0, 0)),
            pl.BlockSpec((1, 128), lambda i: (0, 0)),
            pl.BlockSpec((1, 128), lambda i: (0, 0)),
        ],
        out_specs=node_spec,
        out_shape=jax.ShapeDtypeStruct((N_NODES, 128), jnp.float32),
    )(amax, amin, s1, s2, cnt2d, W, b.reshape(1, 128), g.reshape(1, 128),
      be.reshape(1, 128))


def kernel(x, edge_index, W0, b0, g0, be0, W1, b1, g1, be1, W2, b2, g2, be2):
    loops = jnp.arange(N_NODES, dtype=edge_index.dtype)
    src = jnp.concatenate([edge_index[0], loops])
    dst = jnp.concatenate([edge_index[1], loops])
    dst, src = jax.lax.sort([dst, src], num_keys=1)
    deg = jax.ops.segment_sum(jnp.ones(src.shape[0], jnp.float32), dst,
                              num_segments=N_NODES, indices_are_sorted=True)
    cnt2d = jnp.broadcast_to(deg[:, None], (N_NODES, 128))
    h = x
    for (W, b, g, be) in ((W0, b0, g0, be0), (W1, b1, g1, be1),
                          (W2, b2, g2, be2)):
        m = h[src]
        amax = jax.ops.segment_max(m, dst, num_segments=N_NODES, indices_are_sorted=True)
        amin = -jax.ops.segment_max(-m, dst, num_segments=N_NODES, indices_are_sorted=True)
        s1 = jax.ops.segment_sum(m, dst, num_segments=N_NODES, indices_are_sorted=True)
        s2 = jax.ops.segment_sum(m * m, dst, num_segments=N_NODES, indices_are_sorted=True)
        h = _dense_layer(amax, amin, s1, s2, cnt2d, W, b, g, be)
    return h


# speedup vs baseline: 9.8560x; 9.8560x over previous
import functools

import jax
import jax.numpy as jnp
from jax import lax
from jax.experimental import pallas as pl
from jax.experimental.pallas import tpu as pltpu
from jax.experimental.pallas import tpu_sc as plsc

N_NODES = 10000
N_EDGES = 320000
DELTA = 4.0
ROW_BLK = 400

NW = 32
NPW = 320
RING = 64
EW = 128
E_TOT = N_EDGES + N_NODES
E_PAD = ((E_TOT + EW - 1) // EW) * EW

_NEG = -3.0e38
_POS = 3.0e38


def _sc_body(h_hbm, src_hbm, dst_hbm, bounds_hbm,
             amax_hbm, amin_hbm, s1_hbm, s2_hbm, cnt_hbm,
             idx_v, rows_v, ring0, ring1, ring2, ring3, cnt_v,
             bounds_vm, dstw_vm, gsem):
    w = lax.axis_index("s") * 2 + lax.axis_index("c")
    base = w * NPW
    pltpu.sync_copy(bounds_hbm, bounds_vm)
    lo = bounds_vm[pl.ds(w, 16)][0]
    hi = bounds_vm[pl.ds(w + 1, 16)][0]
    g0 = lo // EW
    g1 = (hi + EW - 1) // EW

    rings = (ring0, ring1, ring2, ring3)
    outs = (amax_hbm, amin_hbm, s1_hbm, s2_hbm)

    ident = (tuple(jnp.full((16,), _NEG, jnp.float32) for _ in range(8))
             + tuple(jnp.full((16,), _POS, jnp.float32) for _ in range(8))
             + tuple(jnp.zeros((16,), jnp.float32) for _ in range(16)))

    def store_segment(ld, ccnt, accs):
        slot = lax.rem(ld, RING)
        for a in range(4):
            for cg in range(8):
                rings[a][pl.ds(slot * 128 + cg * 16, 16)] = accs[a * 8 + cg]
        cnt_v[pl.ds(ld * 16, 16)] = ccnt

        @pl.when(slot == RING - 1)
        def _():
            n0 = pl.multiple_of((base + ld - (RING - 1)) * 128, RING * 128)
            for a in range(4):
                pltpu.sync_copy(rings[a], outs[a].at[pl.ds(n0, RING * 128)])

    def win_body(g, carry):
        estart = pl.multiple_of(g * EW, EW)
        pltpu.sync_copy(src_hbm.at[pl.ds(estart, EW)], idx_v)
        gcp = pltpu.async_copy(h_hbm.at[idx_v], rows_v, gsem)
        pltpu.sync_copy(dst_hbm.at[pl.ds(estart, EW)],
                        dstw_vm.at[pl.ds(0, EW)])
        gcp.wait()
        e_lo = jnp.maximum(lo, estart)
        e_hi = jnp.minimum(hi, estart + EW)

        def edge_body(e, ecarry):
            ew = e - estart
            d_local = dstw_vm[pl.ds(ew, 16)][0] - base
            cur_ld, ccnt = ecarry[0], ecarry[1]
            is_new = d_local != cur_ld

            def close(ops):
                store_segment(ops[0], ops[1], ops[2:])
                return (d_local, jnp.zeros((16,), jnp.float32)) + ident

            def keep(ops):
                return ops

            nc = lax.cond(is_new, close, keep, ecarry)
            cur_ld, ccnt = nc[0], nc[1]
            accs = list(nc[2:])
            for cg in range(8):
                m = rows_v[ew, pl.ds(cg * 16, 16)]
                accs[cg] = jnp.maximum(accs[cg], m)
                accs[8 + cg] = jnp.minimum(accs[8 + cg], m)
                accs[16 + cg] = accs[16 + cg] + m
                accs[24 + cg] = accs[24 + cg] + m * m
            return (cur_ld, ccnt + 1.0) + tuple(accs)

        return lax.fori_loop(e_lo, e_hi, edge_body, carry)

    carry0 = (jnp.int32(0), jnp.zeros((16,), jnp.float32)) + ident
    carry = lax.fori_loop(g0, g1, win_body, carry0)

    store_segment(carry[0], carry[1], carry[2:])

    @pl.when(w == NW - 1)
    def _():
        for a in range(4):
            pltpu.sync_copy(rings[a].at[pl.ds(0, 16 * 128)],
                            outs[a].at[pl.ds((N_NODES - 16) * 128, 16 * 128)])
        pltpu.sync_copy(cnt_v.at[pl.ds(0, 80 * 16)],
                        cnt_hbm.at[pl.ds((NW - 1) * NPW * 16, 80 * 16)])

    @pl.when(w < NW - 1)
    def _():
        pltpu.sync_copy(
            cnt_v, cnt_hbm.at[pl.ds(pl.multiple_of(base * 16, NPW * 16),
                                    NPW * 16)])


@jax.jit
def _sc_aggregate(h, src_p, dst_p, bounds):
    f32 = jnp.float32
    out_type = (jax.ShapeDtypeStruct((N_NODES * 128,), f32),
                jax.ShapeDtypeStruct((N_NODES * 128,), f32),
                jax.ShapeDtypeStruct((N_NODES * 128,), f32),
                jax.ShapeDtypeStruct((N_NODES * 128,), f32),
                jax.ShapeDtypeStruct((N_NODES * 16,), f32))
    scratch = [
        pltpu.VMEM((EW,), jnp.int32),
        pltpu.VMEM((EW, 128), f32),
        pltpu.VMEM((RING * 128,), f32),
        pltpu.VMEM((RING * 128,), f32),
        pltpu.VMEM((RING * 128,), f32),
        pltpu.VMEM((RING * 128,), f32),
        pltpu.VMEM((NPW * 16,), f32),
        pltpu.VMEM((48,), jnp.int32),
        pltpu.VMEM((EW + 16,), jnp.int32),
        pltpu.SemaphoreType.DMA,
    ]
    mesh = plsc.VectorSubcoreMesh(core_axis_name="c", subcore_axis_name="s")
    return pl.kernel(_sc_body, out_type=out_type, mesh=mesh,
                     scratch_types=scratch)(h, src_p, dst_p, bounds)


def _dense_body(amax_ref, amin_ref, s1_ref, s2_ref, cnt_ref, w_ref, b_ref,
                g_ref, be_ref, out_ref):
    amax = amax_ref[...]
    amin = amin_ref[...]
    s1 = s1_ref[...]
    s2 = s2_ref[...]
    cnt = cnt_ref[...]
    mean = s1 / cnt
    var = jnp.maximum(s2 / cnt - mean * mean, 0.0)
    std = jnp.sqrt(var + 1e-5)
    aggs = jnp.concatenate([amax, amin, std, var], axis=1)
    logd = jnp.log(cnt + 1.0)
    t1 = logd / DELTA
    t2 = DELTA / logd
    scaled = jnp.concatenate([aggs, aggs * t1[:, :1], aggs * t2[:, :1]], axis=1)
    h = lax.dot_general(scaled, w_ref[...], (((1,), (0,)), ((), ())),
                        preferred_element_type=jnp.float32)
    h = h + b_ref[...]
    mu = jnp.mean(h, axis=-1, keepdims=True)
    v = jnp.var(h, axis=-1, keepdims=True)
    h = (h - mu) / jnp.sqrt(v + 1e-5) * g_ref[...] + be_ref[...]
    out_ref[...] = jnp.where(h > 0, h, jnp.exp(h) - 1.0)


@jax.jit
def _dense_layer(amax, amin, s1, s2, cnt2d, W, b, g, be):
    grid = (N_NODES // ROW_BLK,)
    node_spec = pl.BlockSpec((ROW_BLK, 128), lambda i: (i, 0))
    return pl.pallas_call(
        _dense_body,
        grid=grid,
        in_specs=[
            node_spec, node_spec, node_spec, node_spec, node_spec,
            pl.BlockSpec((W.shape[0], 128), lambda i: (0, 0)),
            pl.BlockSpec((1, 128), lambda i: (0, 0)),
            pl.BlockSpec((1, 128), lambda i: (0, 0)),
            pl.BlockSpec((1, 128), lambda i: (0, 0)),
        ],
        out_specs=node_spec,
        out_shape=jax.ShapeDtypeStruct((N_NODES, 128), jnp.float32),
    )(amax, amin, s1, s2, cnt2d, W, b.reshape(1, 128), g.reshape(1, 128),
      be.reshape(1, 128))


def kernel(x, edge_index, W0, b0, g0, be0, W1, b1, g1, be1, W2, b2, g2, be2):
    loops = jnp.arange(N_NODES, dtype=edge_index.dtype)
    src = jnp.concatenate([edge_index[0], loops])
    dst = jnp.concatenate([edge_index[1], loops])
    dst_s, src_s = lax.sort([dst, src], num_keys=1)
    src_p = jnp.concatenate(
        [src_s, jnp.zeros((E_PAD - E_TOT,), src_s.dtype)])
    dst_p = jnp.concatenate(
        [dst_s, jnp.full((E_PAD - E_TOT,), 2**30, dst_s.dtype)])
    node_bounds = jnp.arange(NW + 1, dtype=jnp.int32) * NPW
    node_bounds = jnp.minimum(node_bounds, N_NODES)
    bounds = jnp.searchsorted(dst_p, node_bounds, side="left").astype(jnp.int32)
    bounds = jnp.concatenate([bounds, jnp.zeros((15,), jnp.int32)])

    h = x
    cnt2d = None
    for (W, b, g, be) in ((W0, b0, g0, be0), (W1, b1, g1, be1),
                          (W2, b2, g2, be2)):
        amax, amin, s1, s2, cnt = _sc_aggregate(h, src_p, dst_p, bounds)
        amax = amax.reshape(N_NODES, 128)
        amin = amin.reshape(N_NODES, 128)
        s1 = s1.reshape(N_NODES, 128)
        s2 = s2.reshape(N_NODES, 128)
        if cnt2d is None:
            cnt2d = jnp.broadcast_to(cnt.reshape(N_NODES, 16)[:, :1],
                                     (N_NODES, 128))
        h = _dense_layer(amax, amin, s1, s2, cnt2d, W, b, g, be)
    return h
